# Initial kernel scaffold; baseline (speedup 1.0000x reference)
import jax
import jax.numpy as jnp
from jax.experimental import pallas as pl


def kernel(context, center, syn0, syn1_w, syn1_b):
    raise NotImplementedError("write your pallas kernel here")


# speedup vs baseline: 1.3297x; 1.3297x over previous
import functools

import jax
import jax.numpy as jnp
from jax import lax
from jax.experimental import pallas as pl
from jax.experimental.pallas import tpu as pltpu
from jax.experimental.pallas import tpu_sc as plsc


def _tc_body(x_ref, w_ref, b_ref, wc_ref, bcg_ref, cen_ref, out_ref,
             m_ref, s_ref):
    j = pl.program_id(0)
    nv = pl.num_programs(0)

    @pl.when(j == 0)
    def _init():
        m_ref[...] = jnp.full(m_ref.shape, -3.0e38, jnp.float32)
        s_ref[...] = jnp.zeros(s_ref.shape, jnp.float32)

    x = x_ref[...]
    w = w_ref[...]
    logits = lax.dot_general(x, w, (((1,), (1,)), ((), ())),
                             preferred_element_type=jnp.float32)
    logits = logits + b_ref[...]
    m_prev = m_ref[...]
    tmax = jnp.max(logits, axis=1, keepdims=True)
    m_new = jnp.maximum(m_prev, tmax)
    p_sum = jnp.sum(jnp.exp(logits - m_new), axis=1, keepdims=True)
    s_ref[...] = s_ref[...] * jnp.exp(m_prev - m_new) + p_sum
    m_ref[...] = m_new

    @pl.when(j == nv - 1)
    def _fin():
        lse = m_ref[...] + jnp.log(s_ref[...])
        xw = x_ref[...].astype(jnp.float32) * wc_ref[...]
        bg = bcg_ref[...]
        lane = cen_ref[...] & 127
        li = lax.broadcasted_iota(jnp.int32, bg.shape, 1)
        bc = jnp.sum(jnp.where(li == lane, bg, 0.0), axis=1, keepdims=True)
        picked = jnp.sum(xw, axis=1, keepdims=True) + bc
        out_ref[...] = jnp.mean(lse - picked).reshape(1, 1)


def _tc_loss(x16, w16, bpad, wc, bcg, cen2, vt):
    b, d = x16.shape
    vp = w16.shape[0]
    nv = vp // vt
    return pl.pallas_call(
        _tc_body,
        grid=(nv,),
        in_specs=[
            pl.BlockSpec((b, d), lambda j: (0, 0)),
            pl.BlockSpec((vt, d), lambda j: (j, 0)),
            pl.BlockSpec((1, vt), lambda j: (0, j)),
            pl.BlockSpec((b, d), lambda j: (0, 0)),
            pl.BlockSpec((b, 128), lambda j: (0, 0)),
            pl.BlockSpec((b, 1), lambda j: (0, 0)),
        ],
        out_specs=pl.BlockSpec((1, 1), lambda j: (0, 0)),
        out_shape=jax.ShapeDtypeStruct((1, 1), jnp.float32),
        scratch_shapes=[
            pltpu.VMEM((b, 1), jnp.float32),
            pltpu.VMEM((b, 1), jnp.float32),
        ],
        compiler_params=pltpu.CompilerParams(
            dimension_semantics=("arbitrary",)),
    )(x16, w16, bpad, wc, bcg, cen2)


def _make_sc_gather(b, ctx, d, nw):
    rb = b // nw
    sr = 640 // ctx
    nsc = rb // sr
    ng = d // 16
    mesh = plsc.VectorSubcoreMesh(core_axis_name="c", subcore_axis_name="s")

    @functools.partial(
        pl.kernel,
        mesh=mesh,
        out_type=[
            jax.ShapeDtypeStruct((b * d,), jnp.float32),
            jax.ShapeDtypeStruct((b, d), jnp.float32),
            jax.ShapeDtypeStruct((b, 128), jnp.float32),
        ],
        scratch_types=[
            pltpu.VMEM((rb * ctx,), jnp.int32),
            pltpu.VMEM((rb,), jnp.int32),
            pltpu.VMEM((rb,), jnp.int32),
            pltpu.VMEM((640, d), jnp.float32),
            pltpu.VMEM((sr * d,), jnp.float32),
            pltpu.VMEM((rb, d), jnp.float32),
            pltpu.VMEM((rb, 128), jnp.float32),
            pltpu.SemaphoreType.DMA,
            pltpu.SemaphoreType.DMA,
        ],
    )
    def sc(ctx_hbm, cen_hbm, syn0_hbm, w_hbm, b16_hbm,
           emb_out, wc_out, bcg_out,
           idx_v, cidx_v, crow_v, buf, stage, wc_v, bcg_v, sem, sem2):
        wid = lax.axis_index("c") * (nw // 2) + lax.axis_index("s")
        base = wid * rb

        pltpu.sync_copy(ctx_hbm.at[pl.ds(base * ctx, rb * ctx)], idx_v)
        pltpu.sync_copy(cen_hbm.at[pl.ds(base, rb)], cidx_v)

        for q in range(rb // 16):
            crow_v[pl.ds(q * 16, 16)] = lax.shift_right_logical(
                cidx_v[pl.ds(q * 16, 16)], 7)

        cp_w = pltpu.async_copy(w_hbm.at[cidx_v], wc_v, sem2)
        cp_b = pltpu.async_copy(b16_hbm.at[crow_v], bcg_v, sem2)

        def super_chunk(sc_i, _):
            off = sc_i * (sr * ctx)
            cps = [
                pltpu.async_copy(
                    syn0_hbm.at[idx_v.at[pl.ds(off + k * 128, 128)]],
                    buf.at[pl.ds(k * 128, 128)],
                    sem)
                for k in range(5)
            ]
            for cp in cps:
                cp.wait()

            def pool_row(r, _):
                rbase = r * ctx
                for g in range(ng):
                    acc = buf[rbase, pl.ds(g * 16, 16)]
                    for t in range(1, ctx):
                        acc = acc + buf[rbase + t, pl.ds(g * 16, 16)]
                    stage[pl.ds(r * d + g * 16, 16)] = acc * (1.0 / ctx)
                return 0

            lax.fori_loop(0, sr, pool_row, 0)
            pltpu.sync_copy(
                stage, emb_out.at[pl.ds((base + sc_i * sr) * d, sr * d)])
            return 0

        lax.fori_loop(0, nsc, super_chunk, 0)

        cp_w.wait()
        cp_b.wait()
        pltpu.sync_copy(wc_v, wc_out.at[pl.ds(base, rb)])
        pltpu.sync_copy(bcg_v, bcg_out.at[pl.ds(base, rb)])

    return sc


_VT = 512


def kernel(context, center, syn0, syn1_w, syn1_b):
    b, ctx = context.shape
    v, d = syn0.shape
    vt = _VT
    vp = ((v + vt - 1) // vt) * vt

    sc = _make_sc_gather(b, ctx, d, 32)
    emb_flat, wc, bcg = sc(
        context.reshape(-1).astype(jnp.int32),
        center.astype(jnp.int32),
        syn0,
        syn1_w,
        jnp.pad(syn1_b, (0, (-v) % 128)).reshape(-1, 128),
    )

    x16 = emb_flat.reshape(b, d).astype(jnp.bfloat16)
    w16 = jnp.pad(syn1_w, ((0, vp - v), (0, 0))).astype(jnp.bfloat16)
    bpad = jnp.pad(syn1_b, (0, vp - v), constant_values=-1.0e30).reshape(1, vp)
    cen2 = center.astype(jnp.int32).reshape(b, 1)

    loss = _tc_loss(x16, w16, bpad, wc, bcg, cen2, vt)
    return loss[0, 0]


# speedup vs baseline: 4.0033x; 3.0107x over previous
import functools

import jax
import jax.numpy as jnp
from jax import lax
from jax.experimental import pallas as pl
from jax.experimental.pallas import tpu as pltpu
from jax.experimental.pallas import tpu_sc as plsc


_LOG2E = 1.4426950408889634
_LN2 = 0.6931471805599453


def _scan_body(w_ref, b_ref, out_ref, acc_ref):
    j = pl.program_id(0)
    nv = pl.num_programs(0)

    @pl.when(j == 0)
    def _init():
        acc_ref[...] = jnp.full(acc_ref.shape, -3.0e38, jnp.float32)

    w = w_ref[...].astype(jnp.float32)
    rn = jnp.sum(w * w, axis=1, keepdims=True)
    tmax = jnp.max(rn).reshape(1, 1)
    bmax = jnp.max(b_ref[...]).reshape(1, 1)
    prev = acc_ref[...]
    acc_ref[...] = jnp.concatenate(
        [jnp.maximum(prev[:, 0:1], tmax), jnp.maximum(prev[:, 1:2], bmax)],
        axis=1)

    @pl.when(j == nv - 1)
    def _fin():
        a = acc_ref[...]
        wmax = jnp.sqrt(a[:, 0:1]) * 1.05
        out_ref[...] = jnp.concatenate([wmax, a[:, 1:2]], axis=1)


def _w_scan(w16, bpad, vt):
    vp, d = w16.shape
    nv = vp // vt
    return pl.pallas_call(
        _scan_body,
        grid=(nv,),
        in_specs=[
            pl.BlockSpec((vt, d), lambda j: (j, 0)),
            pl.BlockSpec((1, vt), lambda j: (0, j)),
        ],
        out_specs=pl.BlockSpec((1, 2), lambda j: (0, 0)),
        out_shape=jax.ShapeDtypeStruct((1, 2), jnp.float32),
        scratch_shapes=[pltpu.VMEM((1, 2), jnp.float32)],
        compiler_params=pltpu.CompilerParams(
            dimension_semantics=("arbitrary",)),
    )(w16, bpad)


def _tc_body(x_ref, w_ref, b_ref, wc_ref, bcg_ref, cen_ref, wm_ref, out_ref,
             m2n_ref, s_ref):
    j = pl.program_id(0)
    nv = pl.num_programs(0)

    @pl.when(j == 0)
    def _init():
        x = x_ref[...].astype(jnp.float32)
        en = jnp.sqrt(jnp.sum(x * x, axis=1, keepdims=True))
        m2n_ref[...] = -(en * wm_ref[0, 0] + wm_ref[0, 1])
        s_ref[...] = jnp.zeros(s_ref.shape, jnp.float32)

    x = x_ref[...]
    w = w_ref[...]
    l2 = lax.dot_general(x, w, (((1,), (1,)), ((), ())),
                         preferred_element_type=jnp.float32)
    z = l2 + b_ref[...] + m2n_ref[...]
    s_ref[...] += jnp.sum(jnp.exp2(z), axis=1, keepdims=True)

    @pl.when(j == nv - 1)
    def _fin():
        lse2 = jnp.log2(s_ref[...]) - m2n_ref[...]
        xw = x_ref[...].astype(jnp.float32) * wc_ref[...]
        bg = bcg_ref[...]
        lane = cen_ref[...] & 127
        li = lax.broadcasted_iota(jnp.int32, bg.shape, 1)
        bc = jnp.sum(jnp.where(li == lane, bg, 0.0), axis=1, keepdims=True)
        xdot2 = jnp.sum(xw, axis=1, keepdims=True)
        out_ref[...] = jnp.mean(
            _LN2 * (lse2 - xdot2) - bc).reshape(1, 1)


def _tc_loss(x16, w16, b2pad, wc, bcg, cen2, wm, vt):
    b, d = x16.shape
    vp = w16.shape[0]
    nv = vp // vt
    return pl.pallas_call(
        _tc_body,
        grid=(nv,),
        in_specs=[
            pl.BlockSpec((b, d), lambda j: (0, 0)),
            pl.BlockSpec((vt, d), lambda j: (j, 0)),
            pl.BlockSpec((1, vt), lambda j: (0, j)),
            pl.BlockSpec((b, d), lambda j: (0, 0)),
            pl.BlockSpec((b, 128), lambda j: (0, 0)),
            pl.BlockSpec((b, 1), lambda j: (0, 0)),
            pl.BlockSpec((1, 2), lambda j: (0, 0)),
        ],
        out_specs=pl.BlockSpec((1, 1), lambda j: (0, 0)),
        out_shape=jax.ShapeDtypeStruct((1, 1), jnp.float32),
        scratch_shapes=[
            pltpu.VMEM((b, 1), jnp.float32),
            pltpu.VMEM((b, 1), jnp.float32),
        ],
        compiler_params=pltpu.CompilerParams(
            dimension_semantics=("arbitrary",)),
    )(x16, w16, b2pad, wc, bcg, cen2, wm)


def _make_sc_gather(b, ctx, d, nw):
    rb = b // nw
    sr = 640 // ctx
    nsc = rb // sr
    ng = d // 16
    mesh = plsc.VectorSubcoreMesh(core_axis_name="c", subcore_axis_name="s")

    @functools.partial(
        pl.kernel,
        mesh=mesh,
        out_type=[
            jax.ShapeDtypeStruct((b * d,), jnp.float32),
            jax.ShapeDtypeStruct((b, d), jnp.float32),
            jax.ShapeDtypeStruct((b, 128), jnp.float32),
        ],
        scratch_types=[
            pltpu.VMEM((rb * ctx,), jnp.int32),
            pltpu.VMEM((rb,), jnp.int32),
            pltpu.VMEM((rb,), jnp.int32),
            pltpu.VMEM((640, d), jnp.float32),
            pltpu.VMEM((sr * d,), jnp.float32),
            pltpu.VMEM((rb, d), jnp.float32),
            pltpu.VMEM((rb, 128), jnp.float32),
            pltpu.SemaphoreType.DMA,
            pltpu.SemaphoreType.DMA,
        ],
    )
    def sc(ctx_hbm, cen_hbm, syn0_hbm, w_hbm, b16_hbm,
           emb_out, wc_out, bcg_out,
           idx_v, cidx_v, crow_v, buf, stage, wc_v, bcg_v, sem, sem2):
        wid = lax.axis_index("c") * (nw // 2) + lax.axis_index("s")
        base = wid * rb

        pltpu.sync_copy(ctx_hbm.at[pl.ds(base * ctx, rb * ctx)], idx_v)
        pltpu.sync_copy(cen_hbm.at[pl.ds(base, rb)], cidx_v)

        for q in range(rb // 16):
            crow_v[pl.ds(q * 16, 16)] = lax.shift_right_logical(
                cidx_v[pl.ds(q * 16, 16)], 7)

        cp_w = pltpu.async_copy(w_hbm.at[cidx_v], wc_v, sem2)
        cp_b = pltpu.async_copy(b16_hbm.at[crow_v], bcg_v, sem2)

        def super_chunk(sc_i, _):
            off = sc_i * (sr * ctx)
            cps = [
                pltpu.async_copy(
                    syn0_hbm.at[idx_v.at[pl.ds(off + k * 128, 128)]],
                    buf.at[pl.ds(k * 128, 128)],
                    sem)
                for k in range(5)
            ]
            for cp in cps:
                cp.wait()

            def pool_row(r, _):
                rbase = r * ctx
                for g in range(ng):
                    acc = buf[rbase, pl.ds(g * 16, 16)]
                    for t in range(1, ctx):
                        acc = acc + buf[rbase + t, pl.ds(g * 16, 16)]
                    stage[pl.ds(r * d + g * 16, 16)] = acc * (_LOG2E / ctx)
                return 0

            lax.fori_loop(0, sr, pool_row, 0)
            pltpu.sync_copy(
                stage, emb_out.at[pl.ds((base + sc_i * sr) * d, sr * d)])
            return 0

        lax.fori_loop(0, nsc, super_chunk, 0)

        cp_w.wait()
        cp_b.wait()
        pltpu.sync_copy(wc_v, wc_out.at[pl.ds(base, rb)])
        pltpu.sync_copy(bcg_v, bcg_out.at[pl.ds(base, rb)])

    return sc


_VT = 512


def kernel(context, center, syn0, syn1_w, syn1_b):
    b, ctx = context.shape
    v, d = syn0.shape
    vt = _VT
    vp = ((v + vt - 1) // vt) * vt

    sc = _make_sc_gather(b, ctx, d, 32)
    emb_flat, wc, bcg = sc(
        context.reshape(-1).astype(jnp.int32),
        center.astype(jnp.int32),
        syn0,
        syn1_w,
        jnp.pad(syn1_b, (0, (-v) % 128)).reshape(-1, 128),
    )

    x16 = emb_flat.reshape(b, d).astype(jnp.bfloat16)
    w16 = jnp.pad(syn1_w, ((0, vp - v), (0, 0))).astype(jnp.bfloat16)
    b2pad = jnp.pad(syn1_b * _LOG2E, (0, vp - v),
                    constant_values=-1.0e38).reshape(1, vp)
    cen2 = center.astype(jnp.int32).reshape(b, 1)

    wm = _w_scan(w16, b2pad, vt)
    loss = _tc_loss(x16, w16, b2pad, wc, bcg, cen2, wm, vt)
    return loss[0, 0]


# speedup vs baseline: 4.4392x; 1.1089x over previous
import functools

import jax
import jax.numpy as jnp
from jax import lax
from jax.experimental import pallas as pl
from jax.experimental.pallas import tpu as pltpu
from jax.experimental.pallas import tpu_sc as plsc


_LOG2E = 1.4426950408889634
_LN2 = 0.6931471805599453


def _scan_body(w_ref, b_ref, out_ref, acc_ref):
    j = pl.program_id(0)
    nv = pl.num_programs(0)

    @pl.when(j == 0)
    def _init():
        acc_ref[...] = jnp.full(acc_ref.shape, -3.0e38, jnp.float32)

    w = w_ref[...].astype(jnp.float32)
    rn = jnp.sum(w * w, axis=1, keepdims=True)
    tmax = jnp.max(rn).reshape(1, 1)
    bmax = jnp.max(b_ref[...]).reshape(1, 1)
    prev = acc_ref[...]
    acc_ref[...] = jnp.concatenate(
        [jnp.maximum(prev[:, 0:1], tmax), jnp.maximum(prev[:, 1:2], bmax)],
        axis=1)

    @pl.when(j == nv - 1)
    def _fin():
        a = acc_ref[...]
        wmax = jnp.sqrt(a[:, 0:1]) * 1.05
        out_ref[...] = jnp.concatenate([wmax, a[:, 1:2]], axis=1)


def _w_scan(w16, bpad, vt):
    vp, d = w16.shape
    nv = vp // vt
    return pl.pallas_call(
        _scan_body,
        grid=(nv,),
        in_specs=[
            pl.BlockSpec((vt, d), lambda j: (j, 0)),
            pl.BlockSpec((1, vt), lambda j: (0, j)),
        ],
        out_specs=pl.BlockSpec((1, 2), lambda j: (0, 0)),
        out_shape=jax.ShapeDtypeStruct((1, 2), jnp.float32),
        scratch_shapes=[pltpu.VMEM((1, 2), jnp.float32)],
        compiler_params=pltpu.CompilerParams(
            dimension_semantics=("arbitrary",)),
    )(w16, bpad)


def _norm_body(x_ref, wm_ref, out_ref):
    x = x_ref[...].astype(jnp.float32)
    en2 = jnp.max(jnp.sum(x * x, axis=1, keepdims=True))
    beta = jnp.sqrt(en2) * wm_ref[0, 0] + wm_ref[0, 1]
    out_ref[...] = jnp.maximum(beta - 100.0, 0.0).reshape(1, 1)


def _x_norm(x16, wm):
    b, d = x16.shape
    return pl.pallas_call(
        _norm_body,
        out_shape=jax.ShapeDtypeStruct((1, 1), jnp.float32),
    )(x16, wm)


def _tc_body(x_ref, w_ref, b_ref, wc_ref, bcg_ref, cen_ref, m_ref, out_ref,
             s_ref):
    j = pl.program_id(0)
    nv = pl.num_programs(0)
    vt = w_ref.shape[0]

    @pl.when(j == 0)
    def _init():
        s_ref[...] = jnp.zeros(s_ref.shape, jnp.float32)

    x = x_ref[...]
    w = w_ref[...]
    l2 = lax.dot_general(x, w, (((1,), (1,)), ((), ())),
                         preferred_element_type=jnp.float32)
    z = l2 + b_ref[...]
    e = jnp.exp2(z)
    acc = e[:, 0:128]
    for t in range(1, vt // 128):
        acc = acc + e[:, t * 128:(t + 1) * 128]
    s_ref[...] += acc

    @pl.when(j == nv - 1)
    def _fin():
        row = jnp.sum(s_ref[...], axis=1, keepdims=True)
        lse2 = jnp.log2(row) + m_ref[0, 0]
        xw = x_ref[...].astype(jnp.float32) * wc_ref[...]
        bg = bcg_ref[...]
        lane = cen_ref[...] & 127
        li = lax.broadcasted_iota(jnp.int32, bg.shape, 1)
        bc = jnp.sum(jnp.where(li == lane, bg, 0.0), axis=1, keepdims=True)
        xdot2 = jnp.sum(xw, axis=1, keepdims=True)
        out_ref[...] = jnp.mean(
            _LN2 * (lse2 - xdot2) - bc).reshape(1, 1)


def _tc_loss(x16, w16, b2m, wc, bcg, cen2, m, vt):
    b, d = x16.shape
    vp = w16.shape[0]
    nv = vp // vt
    return pl.pallas_call(
        _tc_body,
        grid=(nv,),
        in_specs=[
            pl.BlockSpec((b, d), lambda j: (0, 0)),
            pl.BlockSpec((vt, d), lambda j: (j, 0)),
            pl.BlockSpec((1, vt), lambda j: (0, j)),
            pl.BlockSpec((b, d), lambda j: (0, 0)),
            pl.BlockSpec((b, 128), lambda j: (0, 0)),
            pl.BlockSpec((b, 1), lambda j: (0, 0)),
            pl.BlockSpec((1, 1), lambda j: (0, 0)),
        ],
        out_specs=pl.BlockSpec((1, 1), lambda j: (0, 0)),
        out_shape=jax.ShapeDtypeStruct((1, 1), jnp.float32),
        scratch_shapes=[
            pltpu.VMEM((b, 128), jnp.float32),
        ],
        compiler_params=pltpu.CompilerParams(
            dimension_semantics=("arbitrary",)),
    )(x16, w16, b2m, wc, bcg, cen2, m)


def _make_sc_gather(b, ctx, d, nw):
    rb = b // nw
    sr = 640 // ctx
    nsc = rb // sr
    ng = d // 16
    mesh = plsc.VectorSubcoreMesh(core_axis_name="c", subcore_axis_name="s")

    @functools.partial(
        pl.kernel,
        mesh=mesh,
        out_type=[
            jax.ShapeDtypeStruct((b * d,), jnp.float32),
            jax.ShapeDtypeStruct((b, d), jnp.float32),
            jax.ShapeDtypeStruct((b, 128), jnp.float32),
        ],
        scratch_types=[
            pltpu.VMEM((rb * ctx,), jnp.int32),
            pltpu.VMEM((rb,), jnp.int32),
            pltpu.VMEM((rb,), jnp.int32),
            pltpu.VMEM((640, d), jnp.float32),
            pltpu.VMEM((sr * d,), jnp.float32),
            pltpu.VMEM((rb, d), jnp.float32),
            pltpu.VMEM((rb, 128), jnp.float32),
            pltpu.SemaphoreType.DMA,
            pltpu.SemaphoreType.DMA,
        ],
    )
    def sc(ctx_hbm, cen_hbm, syn0_hbm, w_hbm, b16_hbm,
           emb_out, wc_out, bcg_out,
           idx_v, cidx_v, crow_v, buf, stage, wc_v, bcg_v, sem, sem2):
        wid = lax.axis_index("c") * (nw // 2) + lax.axis_index("s")
        base = wid * rb

        pltpu.sync_copy(ctx_hbm.at[pl.ds(base * ctx, rb * ctx)], idx_v)
        pltpu.sync_copy(cen_hbm.at[pl.ds(base, rb)], cidx_v)

        for q in range(rb // 16):
            crow_v[pl.ds(q * 16, 16)] = lax.shift_right_logical(
                cidx_v[pl.ds(q * 16, 16)], 7)

        cp_w = pltpu.async_copy(w_hbm.at[cidx_v], wc_v, sem2)
        cp_b = pltpu.async_copy(b16_hbm.at[crow_v], bcg_v, sem2)

        def super_chunk(sc_i, _):
            off = sc_i * (sr * ctx)
            cps = [
                pltpu.async_copy(
                    syn0_hbm.at[idx_v.at[pl.ds(off + k * 128, 128)]],
                    buf.at[pl.ds(k * 128, 128)],
                    sem)
                for k in range(5)
            ]
            for cp in cps:
                cp.wait()

            def pool_row(r, _):
                rbase = r * ctx
                for g in range(ng):
                    acc = buf[rbase, pl.ds(g * 16, 16)]
                    for t in range(1, ctx):
                        acc = acc + buf[rbase + t, pl.ds(g * 16, 16)]
                    stage[pl.ds(r * d + g * 16, 16)] = acc * (_LOG2E / ctx)
                return 0

            lax.fori_loop(0, sr, pool_row, 0)
            pltpu.sync_copy(
                stage, emb_out.at[pl.ds((base + sc_i * sr) * d, sr * d)])
            return 0

        lax.fori_loop(0, nsc, super_chunk, 0)

        cp_w.wait()
        cp_b.wait()
        pltpu.sync_copy(wc_v, wc_out.at[pl.ds(base, rb)])
        pltpu.sync_copy(bcg_v, bcg_out.at[pl.ds(base, rb)])

    return sc


_VT = 512


def kernel(context, center, syn0, syn1_w, syn1_b):
    b, ctx = context.shape
    v, d = syn0.shape
    vt = _VT
    vp = ((v + vt - 1) // vt) * vt

    sc = _make_sc_gather(b, ctx, d, 32)
    emb_flat, wc, bcg = sc(
        context.reshape(-1).astype(jnp.int32),
        center.astype(jnp.int32),
        syn0,
        syn1_w,
        jnp.pad(syn1_b, (0, (-v) % 128)).reshape(-1, 128),
    )

    x16 = emb_flat.reshape(b, d).astype(jnp.bfloat16)
    w16 = jnp.pad(syn1_w, ((0, vp - v), (0, 0))).astype(jnp.bfloat16)
    b2pad = jnp.pad(syn1_b * _LOG2E, (0, vp - v),
                    constant_values=-1.0e38).reshape(1, vp)
    cen2 = center.astype(jnp.int32).reshape(b, 1)

    wm = _w_scan(w16, b2pad, vt)
    m = _x_norm(x16, wm)
    b2m = b2pad - m[0, 0]
    loss = _tc_loss(x16, w16, b2m, wc, bcg, cen2, m, vt)
    return loss[0, 0]


# speedup vs baseline: 5.3608x; 1.2076x over previous
import functools

import jax
import jax.numpy as jnp
from jax import lax
from jax.experimental import pallas as pl
from jax.experimental.pallas import tpu as pltpu
from jax.experimental.pallas import tpu_sc as plsc


_LOG2E = 1.4426950408889634
_LN2 = 0.6931471805599453


def _scan_body(w_ref, b_ref, out_ref, acc_ref):
    j = pl.program_id(0)
    nv = pl.num_programs(0)

    @pl.when(j == 0)
    def _init():
        acc_ref[...] = jnp.full(acc_ref.shape, -3.0e38, jnp.float32)

    w = w_ref[...].astype(jnp.float32)
    rn = jnp.sum(w * w, axis=1, keepdims=True)
    tmax = jnp.max(rn).reshape(1, 1)
    bmax = jnp.max(b_ref[...]).reshape(1, 1)
    prev = acc_ref[...]
    acc_ref[...] = jnp.concatenate(
        [jnp.maximum(prev[:, 0:1], tmax), jnp.maximum(prev[:, 1:2], bmax)],
        axis=1)

    @pl.when(j == nv - 1)
    def _fin():
        a = acc_ref[...]
        wmax = jnp.sqrt(a[:, 0:1]) * 1.05
        out_ref[...] = jnp.concatenate([wmax, a[:, 1:2]], axis=1)


def _w_scan(w16, bpad, vt):
    vp, d = w16.shape
    nv = vp // vt
    return pl.pallas_call(
        _scan_body,
        grid=(nv,),
        in_specs=[
            pl.BlockSpec((vt, d), lambda j: (j, 0)),
            pl.BlockSpec((1, vt), lambda j: (0, j)),
        ],
        out_specs=pl.BlockSpec((1, 2), lambda j: (0, 0)),
        out_shape=jax.ShapeDtypeStruct((1, 2), jnp.float32),
        scratch_shapes=[pltpu.VMEM((1, 2), jnp.float32)],
        compiler_params=pltpu.CompilerParams(
            dimension_semantics=("arbitrary",)),
    )(w16, bpad)


def _norm_body(x_ref, wm_ref, out_ref):
    x = x_ref[...].astype(jnp.float32)
    en2 = jnp.max(jnp.sum(x * x, axis=1, keepdims=True))
    beta = jnp.sqrt(en2) * wm_ref[0, 0] + wm_ref[0, 1]
    out_ref[...] = jnp.maximum(beta - 100.0, 0.0).reshape(1, 1)


def _x_norm(x16, wm):
    b, d = x16.shape
    return pl.pallas_call(
        _norm_body,
        out_shape=jax.ShapeDtypeStruct((1, 1), jnp.float32),
    )(x16, wm)


def _tc_body(x_ref, w_ref, b_ref, wc_ref, bcg_ref, cen_ref, m_ref, out_ref,
             s_ref):
    j = pl.program_id(0)
    nv = pl.num_programs(0)
    vt = w_ref.shape[0]

    @pl.when(j == 0)
    def _init():
        s_ref[...] = jnp.zeros(s_ref.shape, jnp.float32)

    x = x_ref[...]
    w = w_ref[...]
    l2 = lax.dot_general(x, w, (((1,), (1,)), ((), ())),
                         preferred_element_type=jnp.float32)
    z = l2 + b_ref[...]
    e = jnp.exp2(z)
    acc = e[:, 0:128]
    for t in range(1, vt // 128):
        acc = acc + e[:, t * 128:(t + 1) * 128]
    s_ref[...] += acc

    @pl.when(j == nv - 1)
    def _fin():
        row = jnp.sum(s_ref[...], axis=1, keepdims=True)
        lse2 = jnp.log2(row) + m_ref[0, 0]
        xw = x_ref[...].astype(jnp.float32) * wc_ref[...]
        bg = bcg_ref[...]
        lane = cen_ref[...] & 127
        li = lax.broadcasted_iota(jnp.int32, bg.shape, 1)
        bc = jnp.sum(jnp.where(li == lane, bg, 0.0), axis=1, keepdims=True)
        xdot2 = jnp.sum(xw, axis=1, keepdims=True)
        out_ref[...] = jnp.mean(
            _LN2 * (lse2 - xdot2) - bc).reshape(1, 1)


def _tc_loss(x16, w16, b2m, wc, bcg, cen2, m, vt):
    b, d = x16.shape
    vp = w16.shape[0]
    nv = vp // vt
    return pl.pallas_call(
        _tc_body,
        grid=(nv,),
        in_specs=[
            pl.BlockSpec((b, d), lambda j: (0, 0)),
            pl.BlockSpec((vt, d), lambda j: (j, 0)),
            pl.BlockSpec((1, vt), lambda j: (0, j)),
            pl.BlockSpec((b, d), lambda j: (0, 0)),
            pl.BlockSpec((b, 128), lambda j: (0, 0)),
            pl.BlockSpec((b, 1), lambda j: (0, 0)),
            pl.BlockSpec((1, 1), lambda j: (0, 0)),
        ],
        out_specs=pl.BlockSpec((1, 1), lambda j: (0, 0)),
        out_shape=jax.ShapeDtypeStruct((1, 1), jnp.float32),
        scratch_shapes=[
            pltpu.VMEM((b, 128), jnp.float32),
        ],
        compiler_params=pltpu.CompilerParams(
            dimension_semantics=("arbitrary",)),
    )(x16, w16, b2m, wc, bcg, cen2, m)


def _make_sc_gather(b, ctx, d, nw):
    rb = b // nw
    sr = 640 // ctx
    nsc = rb // sr
    ng = d // 16
    mesh = plsc.VectorSubcoreMesh(core_axis_name="c", subcore_axis_name="s")

    @functools.partial(
        pl.kernel,
        mesh=mesh,
        out_type=[
            jax.ShapeDtypeStruct((b * d,), jnp.float32),
            jax.ShapeDtypeStruct((b, d), jnp.float32),
            jax.ShapeDtypeStruct((b, 128), jnp.float32),
        ],
        scratch_types=[
            pltpu.VMEM((rb * ctx,), jnp.int32),
            pltpu.VMEM((rb,), jnp.int32),
            pltpu.VMEM((rb,), jnp.int32),
            pltpu.VMEM((640, d), jnp.float32),
            pltpu.VMEM((sr * d,), jnp.float32),
            pltpu.VMEM((rb, d), jnp.float32),
            pltpu.VMEM((rb, 128), jnp.float32),
            pltpu.SemaphoreType.DMA,
            pltpu.SemaphoreType.DMA,
        ],
    )
    def sc(ctx_hbm, cen_hbm, syn0_hbm, w_hbm, b16_hbm,
           emb_out, wc_out, bcg_out,
           idx_v, cidx_v, crow_v, buf, stage, wc_v, bcg_v, sem, sem2):
        wid = lax.axis_index("c") * (nw // 2) + lax.axis_index("s")
        base = wid * rb

        pltpu.sync_copy(ctx_hbm.at[pl.ds(base * ctx, rb * ctx)], idx_v)
        pltpu.sync_copy(cen_hbm.at[pl.ds(base, rb)], cidx_v)

        for q in range(rb // 16):
            crow_v[pl.ds(q * 16, 16)] = lax.shift_right_logical(
                cidx_v[pl.ds(q * 16, 16)], 7)

        cp_w = pltpu.async_copy(w_hbm.at[cidx_v], wc_v, sem2)
        cp_b = pltpu.async_copy(b16_hbm.at[crow_v], bcg_v, sem2)

        def super_chunk(sc_i, _):
            off = sc_i * (sr * ctx)
            cps = [
                pltpu.async_copy(
                    syn0_hbm.at[idx_v.at[pl.ds(off + k * 128, 128)]],
                    buf.at[pl.ds(k * 128, 128)],
                    sem)
                for k in range(5)
            ]
            for cp in cps:
                cp.wait()

            def pool_row(r, _):
                rbase = r * ctx
                for g in range(ng):
                    acc = buf[rbase, pl.ds(g * 16, 16)]
                    for t in range(1, ctx):
                        acc = acc + buf[rbase + t, pl.ds(g * 16, 16)]
                    stage[pl.ds(r * d + g * 16, 16)] = acc * (_LOG2E / ctx)
                return 0

            lax.fori_loop(0, sr, pool_row, 0)
            pltpu.sync_copy(
                stage, emb_out.at[pl.ds((base + sc_i * sr) * d, sr * d)])
            return 0

        lax.fori_loop(0, nsc, super_chunk, 0)

        cp_w.wait()
        cp_b.wait()
        pltpu.sync_copy(wc_v, wc_out.at[pl.ds(base, rb)])
        pltpu.sync_copy(bcg_v, bcg_out.at[pl.ds(base, rb)])

    return sc


_VT = 1024


def kernel(context, center, syn0, syn1_w, syn1_b):
    b, ctx = context.shape
    v, d = syn0.shape
    vt = _VT
    vp = ((v + vt - 1) // vt) * vt

    sc = _make_sc_gather(b, ctx, d, 32)
    emb_flat, wc, bcg = sc(
        context.reshape(-1).astype(jnp.int32),
        center.astype(jnp.int32),
        syn0,
        syn1_w,
        jnp.pad(syn1_b, (0, (-v) % 128)).reshape(-1, 128),
    )

    x16 = emb_flat.reshape(b, d).astype(jnp.bfloat16)
    w16 = jnp.pad(syn1_w, ((0, vp - v), (0, 0))).astype(jnp.bfloat16)
    b2pad = jnp.pad(syn1_b * _LOG2E, (0, vp - v),
                    constant_values=-1.0e38).reshape(1, vp)
    cen2 = center.astype(jnp.int32).reshape(b, 1)

    wm = _w_scan(w16, b2pad, vt)
    m = _x_norm(x16, wm)
    b2m = b2pad - m[0, 0]
    loss = _tc_loss(x16, w16, b2m, wc, bcg, cen2, m, vt)
    return loss[0, 0]


# speedup vs baseline: 5.9668x; 1.1130x over previous
import functools

import jax
import jax.numpy as jnp
from jax import lax
from jax.experimental import pallas as pl
from jax.experimental.pallas import tpu as pltpu
from jax.experimental.pallas import tpu_sc as plsc


_LOG2E = 1.4426950408889634
_LN2 = 0.6931471805599453


def _scan_body(w_ref, b_ref, out_ref, acc_ref):
    j = pl.program_id(0)
    nv = pl.num_programs(0)

    @pl.when(j == 0)
    def _init():
        acc_ref[...] = jnp.full(acc_ref.shape, -3.0e38, jnp.float32)

    w = w_ref[...].astype(jnp.float32)
    rn = jnp.sum(w * w, axis=1, keepdims=True)
    tmax = jnp.max(rn).reshape(1, 1)
    bmax = jnp.max(b_ref[...]).reshape(1, 1)
    prev = acc_ref[...]
    acc_ref[...] = jnp.concatenate(
        [jnp.maximum(prev[:, 0:1], tmax), jnp.maximum(prev[:, 1:2], bmax)],
        axis=1)

    @pl.when(j == nv - 1)
    def _fin():
        a = acc_ref[...]
        wmax = jnp.sqrt(a[:, 0:1]) * 1.05
        out_ref[...] = jnp.concatenate([wmax, a[:, 1:2]], axis=1)


def _w_scan(w16, bpad, vt):
    vp, d = w16.shape
    nv = vp // vt
    return pl.pallas_call(
        _scan_body,
        grid=(nv,),
        in_specs=[
            pl.BlockSpec((vt, d), lambda j: (j, 0)),
            pl.BlockSpec((1, vt), lambda j: (0, j)),
        ],
        out_specs=pl.BlockSpec((1, 2), lambda j: (0, 0)),
        out_shape=jax.ShapeDtypeStruct((1, 2), jnp.float32),
        scratch_shapes=[pltpu.VMEM((1, 2), jnp.float32)],
        compiler_params=pltpu.CompilerParams(
            dimension_semantics=("arbitrary",)),
    )(w16, bpad)


def _norm_body(x_ref, wm_ref, out_ref):
    x = x_ref[...].astype(jnp.float32)
    en2 = jnp.max(jnp.sum(x * x, axis=1, keepdims=True))
    beta = jnp.sqrt(en2) * wm_ref[0, 0] + wm_ref[0, 1]
    out_ref[...] = jnp.maximum(beta - 100.0, 0.0).reshape(1, 1)


def _x_norm(x16, wm):
    b, d = x16.shape
    return pl.pallas_call(
        _norm_body,
        out_shape=jax.ShapeDtypeStruct((1, 1), jnp.float32),
    )(x16, wm)


def _tc_body(x_ref, w_ref, b_ref, wc_ref, bcg_ref, cen_ref, m_ref, out_ref,
             s_ref):
    j = pl.program_id(0)
    nv = pl.num_programs(0)
    vt = w_ref.shape[0]

    @pl.when(j == 0)
    def _init():
        s_ref[...] = jnp.zeros(s_ref.shape, jnp.float32)

    x = x_ref[...]
    w = w_ref[...]
    l2 = lax.dot_general(x, w, (((1,), (1,)), ((), ())),
                         preferred_element_type=jnp.float32)
    z = l2 + b_ref[...]
    e = jnp.exp2(z)
    acc = e[:, 0:128]
    for t in range(1, vt // 128):
        acc = acc + e[:, t * 128:(t + 1) * 128]
    s_ref[...] += acc

    @pl.when(j == nv - 1)
    def _fin():
        row = jnp.sum(s_ref[...], axis=1, keepdims=True)
        lse2 = jnp.log2(row) + m_ref[0, 0]
        xw = x_ref[...].astype(jnp.float32) * wc_ref[...]
        bg = bcg_ref[...]
        lane = cen_ref[...] & 127
        li = lax.broadcasted_iota(jnp.int32, bg.shape, 1)
        bc = jnp.sum(jnp.where(li == lane, bg, 0.0), axis=1, keepdims=True)
        xdot2 = jnp.sum(xw, axis=1, keepdims=True)
        out_ref[...] = jnp.mean(
            _LN2 * (lse2 - xdot2) - bc).reshape(1, 1)


def _tc_loss(x16, w16, b2m, wc, bcg, cen2, m, vt):
    b, d = x16.shape
    vp = w16.shape[0]
    nv = vp // vt
    return pl.pallas_call(
        _tc_body,
        grid=(nv,),
        in_specs=[
            pl.BlockSpec((b, d), lambda j: (0, 0)),
            pl.BlockSpec((vt, d), lambda j: (j, 0)),
            pl.BlockSpec((1, vt), lambda j: (0, j)),
            pl.BlockSpec((b, d), lambda j: (0, 0)),
            pl.BlockSpec((b, 128), lambda j: (0, 0)),
            pl.BlockSpec((b, 1), lambda j: (0, 0)),
            pl.BlockSpec((1, 1), lambda j: (0, 0)),
        ],
        out_specs=pl.BlockSpec((1, 1), lambda j: (0, 0)),
        out_shape=jax.ShapeDtypeStruct((1, 1), jnp.float32),
        scratch_shapes=[
            pltpu.VMEM((b, 128), jnp.float32),
        ],
        compiler_params=pltpu.CompilerParams(
            dimension_semantics=("arbitrary",)),
    )(x16, w16, b2m, wc, bcg, cen2, m)


def _make_sc_gather(b, ctx, d, nw):
    rb = b // nw
    sr = 640 // ctx
    nsc = rb // sr
    ng = d // 16
    mesh = plsc.VectorSubcoreMesh(core_axis_name="c", subcore_axis_name="s")

    @functools.partial(
        pl.kernel,
        mesh=mesh,
        out_type=[
            jax.ShapeDtypeStruct((b * d,), jnp.float32),
            jax.ShapeDtypeStruct((b, d), jnp.float32),
            jax.ShapeDtypeStruct((b, 128), jnp.float32),
        ],
        scratch_types=[
            pltpu.VMEM((rb * ctx,), jnp.int32),
            pltpu.VMEM((rb,), jnp.int32),
            pltpu.VMEM((rb,), jnp.int32),
            pltpu.VMEM((640, d), jnp.float32),
            pltpu.VMEM((sr * d,), jnp.float32),
            pltpu.VMEM((rb, d), jnp.float32),
            pltpu.VMEM((rb, 128), jnp.float32),
            pltpu.SemaphoreType.DMA,
            pltpu.SemaphoreType.DMA,
        ],
    )
    def sc(ctx_hbm, cen_hbm, syn0_hbm, w_hbm, b16_hbm,
           emb_out, wc_out, bcg_out,
           idx_v, cidx_v, crow_v, buf, stage, wc_v, bcg_v, sem, sem2):
        wid = lax.axis_index("c") * (nw // 2) + lax.axis_index("s")
        base = wid * rb

        pltpu.sync_copy(ctx_hbm.at[pl.ds(base * ctx, rb * ctx)], idx_v)
        pltpu.sync_copy(cen_hbm.at[pl.ds(base, rb)], cidx_v)

        for q in range(rb // 16):
            crow_v[pl.ds(q * 16, 16)] = lax.shift_right_logical(
                cidx_v[pl.ds(q * 16, 16)], 7)

        cp_w = pltpu.async_copy(w_hbm.at[cidx_v], wc_v, sem2)
        cp_b = pltpu.async_copy(b16_hbm.at[crow_v], bcg_v, sem2)

        def super_chunk(sc_i, _):
            off = sc_i * (sr * ctx)
            cps = [
                pltpu.async_copy(
                    syn0_hbm.at[idx_v.at[pl.ds(off + k * 128, 128)]],
                    buf.at[pl.ds(k * 128, 128)],
                    sem)
                for k in range(5)
            ]
            for cp in cps:
                cp.wait()

            def pool_row(r, _):
                rbase = r * ctx
                for g in range(ng):
                    acc = buf[rbase, pl.ds(g * 16, 16)]
                    for t in range(1, ctx):
                        acc = acc + buf[rbase + t, pl.ds(g * 16, 16)]
                    stage[pl.ds(r * d + g * 16, 16)] = acc * (_LOG2E / ctx)
                return 0

            lax.fori_loop(0, sr, pool_row, 0)
            pltpu.sync_copy(
                stage, emb_out.at[pl.ds((base + sc_i * sr) * d, sr * d)])
            return 0

        lax.fori_loop(0, nsc, super_chunk, 0)

        cp_w.wait()
        cp_b.wait()
        pltpu.sync_copy(wc_v, wc_out.at[pl.ds(base, rb)])
        pltpu.sync_copy(bcg_v, bcg_out.at[pl.ds(base, rb)])

    return sc


_VT = 2048


def kernel(context, center, syn0, syn1_w, syn1_b):
    b, ctx = context.shape
    v, d = syn0.shape
    vt = _VT
    vp = ((v + vt - 1) // vt) * vt

    sc = _make_sc_gather(b, ctx, d, 32)
    emb_flat, wc, bcg = sc(
        context.reshape(-1).astype(jnp.int32),
        center.astype(jnp.int32),
        syn0,
        syn1_w,
        jnp.pad(syn1_b, (0, (-v) % 128)).reshape(-1, 128),
    )

    x16 = emb_flat.reshape(b, d).astype(jnp.bfloat16)
    w16 = jnp.pad(syn1_w, ((0, vp - v), (0, 0))).astype(jnp.bfloat16)
    b2pad = jnp.pad(syn1_b * _LOG2E, (0, vp - v),
                    constant_values=-1.0e38).reshape(1, vp)
    cen2 = center.astype(jnp.int32).reshape(b, 1)

    wm = _w_scan(w16, b2pad, vt)
    m = _x_norm(x16, wm)
    b2m = b2pad - m[0, 0]
    loss = _tc_loss(x16, w16, b2m, wc, bcg, cen2, m, vt)
    return loss[0, 0]
